# baseline (speedup 1.0000x reference)
import functools

import jax
import jax.numpy as jnp
from jax import lax
from jax.experimental import pallas as pl
from jax.experimental.pallas import tpu as pltpu
from jax.experimental.pallas import tpu_sc as plsc

N = 10000
E = 320000
D = 128
D_EDGE = 16
EPS = 1e-5

NC = 2
NS = 16
NW = NC * NS
E_PAD = 327680
NSLICE = 2
E_SL = E_PAD // NSLICE
EPW = E_SL // NW
G = 128
NCH_G = EPW // G
DEPTH = 2
GS = 128
NCH_S = EPW // GS
N_PAD = 10240
RPS = N_PAD // NS

_MESH = plsc.VectorSubcoreMesh(core_axis_name="c", subcore_axis_name="s")


def _pack_bf16_pairs(t):
    bh = lax.bitcast_convert_type(t[:, :D], jnp.int32)
    bl = lax.bitcast_convert_type(t[:, D:], jnp.int32)
    rh = bh + jnp.int32(0x7FFF) + ((bh >> 16) & 1)
    rl = bl + jnp.int32(0x7FFF) + ((bl >> 16) & 1)
    return (rh & jnp.int32(-65536)) | lax.shift_right_logical(rl, 16)


def _unpack_hi(p):
    return lax.bitcast_convert_type(p & jnp.int32(-65536), jnp.float32)


def _unpack_lo(p):
    return lax.bitcast_convert_type(p << 16, jnp.float32)


def _proj_body(x_ref, wd_ref, ws_ref, tdst_ref, tsrc_ref):
    x = x_ref[...]
    tdst_ref[...] = _pack_bf16_pairs(
        jnp.dot(x, wd_ref[...], preferred_element_type=jnp.float32))
    tsrc_ref[...] = _pack_bf16_pairs(
        jnp.dot(x, ws_ref[...], preferred_element_type=jnp.float32))


def _project(x, wd, wsrc):
    return pl.pallas_call(
        _proj_body,
        out_shape=(
            jax.ShapeDtypeStruct((N, D), jnp.int32),
            jax.ShapeDtypeStruct((N, D), jnp.int32),
        ),
    )(x, wd, wsrc)


def _gather(tdst, tsrc, dst_idx, src_idx):
    out_ty = (
        jax.ShapeDtypeStruct((E_SL, D), jnp.int32),
        jax.ShapeDtypeStruct((E_SL, D), jnp.int32),
    )

    @functools.partial(
        pl.kernel,
        out_type=out_ty,
        mesh=_MESH,
        scratch_types=(
            [pltpu.VMEM((EPW,), jnp.int32)] * 2
            + [pltpu.VMEM((G, D), jnp.int32)] * (2 * DEPTH)
            + [pltpu.SemaphoreType.DMA] * (2 * DEPTH)
        ),
    )
    def k(tdst_h, tsrc_h, di_h, si_h, a_h, b_h, idxd, idxs, *rest):
        bd = list(rest[0:DEPTH])
        bs = list(rest[DEPTH:2 * DEPTH])
        gsem = list(rest[2 * DEPTH:3 * DEPTH])
        wsem = list(rest[3 * DEPTH:4 * DEPTH])
        cid = lax.axis_index("c")
        sid = lax.axis_index("s")
        wid = sid * NC + cid
        base = wid * EPW
        pltpu.sync_copy(di_h.at[pl.ds(base, EPW)], idxd)
        pltpu.sync_copy(si_h.at[pl.ds(base, EPW)], idxs)

        def gstart(c, b):
            pltpu.async_copy(tdst_h.at[idxd.at[pl.ds(c * G, G)]], bd[b], gsem[b])
            pltpu.async_copy(tsrc_h.at[idxs.at[pl.ds(c * G, G)]], bs[b], gsem[b])

        def gwait(c, b):
            pltpu.make_async_copy(
                tdst_h.at[idxd.at[pl.ds(c * G, G)]], bd[b], gsem[b]).wait()
            pltpu.make_async_copy(
                tsrc_h.at[idxs.at[pl.ds(c * G, G)]], bs[b], gsem[b]).wait()

        def wstart(c, b):
            pltpu.async_copy(bd[b], a_h.at[pl.ds(base + c * G, G)], wsem[b])
            pltpu.async_copy(bs[b], b_h.at[pl.ds(base + c * G, G)], wsem[b])

        def wwait(c, b):
            pltpu.make_async_copy(
                bd[b], a_h.at[pl.ds(base + c * G, G)], wsem[b]).wait()
            pltpu.make_async_copy(
                bs[b], b_h.at[pl.ds(base + c * G, G)], wsem[b]).wait()

        for b in range(DEPTH):
            gstart(b, b)

        @pl.loop(0, NCH_G, step=DEPTH)
        def _(cc):
            for b in range(DEPTH):
                c = cc + b
                gwait(c, b)
                wstart(c, b)

                @pl.when(c + DEPTH < NCH_G)
                def _():
                    wwait(c, b)
                    gstart(c + DEPTH, b)

        for b in range(DEPTH):
            wwait(NCH_G - DEPTH + b, b)

    return k(tdst, tsrc, dst_idx, src_idx)


BE = 2048


def _mlp_body(base, a_ref, b_ref, ea_ref, wfe_ref, wse_ref, bf_ref, bs_ref,
              msg_ref):
    a = a_ref[...]
    b = b_ref[...]
    ea = ea_ref[...]
    ef = jnp.dot(ea, wfe_ref[...], preferred_element_type=jnp.float32)
    es = jnp.dot(ea, wse_ref[...], preferred_element_type=jnp.float32)
    zf = _unpack_hi(a) + _unpack_hi(b) + ef + bf_ref[...]
    zs = _unpack_lo(a) + _unpack_lo(b) + es + bs_ref[...]
    gate = jax.nn.sigmoid(zf)
    val = jnp.maximum(zs, 0.0) + jnp.log1p(jnp.exp(-jnp.abs(zs)))
    msg = gate * val
    row = (base + pl.program_id(0) * BE
           + lax.broadcasted_iota(jnp.int32, (BE, D), 0))
    msg_ref[...] = jnp.where(row < E, msg, 0.0)


def _mlp(base, a, b, ea, wfe, wse, bf, bs):
    nblk = E_SL // BE
    return pl.pallas_call(
        functools.partial(_mlp_body, base),
        grid=(nblk,),
        in_specs=[
            pl.BlockSpec((BE, D), lambda i: (i, 0)),
            pl.BlockSpec((BE, D), lambda i: (i, 0)),
            pl.BlockSpec((BE, D_EDGE), lambda i: (i, 0)),
            pl.BlockSpec((D_EDGE, D), lambda i: (0, 0)),
            pl.BlockSpec((D_EDGE, D), lambda i: (0, 0)),
            pl.BlockSpec((1, D), lambda i: (0, 0)),
            pl.BlockSpec((1, D), lambda i: (0, 0)),
        ],
        out_specs=pl.BlockSpec((BE, D), lambda i: (i, 0)),
        out_shape=jax.ShapeDtypeStruct((E_SL, D), jnp.float32),
        compiler_params=pltpu.CompilerParams(
            dimension_semantics=("parallel",)),
    )(a, b, ea, wfe, wse, bf, bs)


def _scatter(msgs, dst2d, zeros):
    out_ty = jax.ShapeDtypeStruct((NC, N_PAD, D), jnp.float32)

    IW = ((NCH_S + 7) // 8) * 8 + 8

    @functools.partial(
        pl.kernel,
        out_type=out_ty,
        mesh=_MESH,
        scratch_types=[
            pltpu.VMEM((NSLICE * IW, GS), jnp.int32),
            pltpu.VMEM((GS, D), jnp.float32),
            pltpu.VMEM((GS, D), jnp.float32),
            pltpu.VMEM_SHARED((N_PAD, D), jnp.float32),
            pltpu.SemaphoreType.DMA,
            pltpu.SemaphoreType.DMA,
        ],
    )
    def k(*refs):
        msg_hs = list(refs[:NSLICE])
        (dst_h, z_h, out_h, idx2, m0, m1, agg, s0, s1) = refs[NSLICE:]
        cid = lax.axis_index("c")
        sid = lax.axis_index("s")
        wid = sid * NC + cid
        offs = []
        for s in range(NSLICE):
            gstart = s * (E_SL // GS) + wid * NCH_S
            astart = (gstart // 8) * 8
            offs.append(gstart - astart)
            pltpu.sync_copy(dst_h.at[pl.ds(astart, IW)],
                            idx2.at[pl.ds(s * IW, IW)])
        pltpu.sync_copy(z_h.at[pl.ds(sid * RPS, RPS)],
                        agg.at[pl.ds(sid * RPS, RPS)])
        plsc.subcore_barrier()

        mbuf = [m0, m1]
        sem = [s0, s1]
        SD = 2

        for s in range(NSLICE):
            msg_h = msg_hs[s]
            ebase = wid * EPW

            def mstart(c, b):
                pltpu.async_copy(
                    msg_h.at[pl.ds(ebase + c * GS, GS)], mbuf[b], sem[b])

            def mwait(c, b):
                pltpu.make_async_copy(
                    msg_h.at[pl.ds(ebase + c * GS, GS)], mbuf[b], sem[b]).wait()

            for b in range(SD):
                mstart(b, b)

            @pl.loop(0, NCH_S, step=SD)
            def _(cc):
                for b in range(SD):
                    c = cc + b
                    mwait(c, b)
                    pltpu.sync_copy(mbuf[b],
                                    agg.at[idx2.at[s * IW + offs[s] + c]],
                                    add=True)

                    @pl.when(c + SD < NCH_S)
                    def _():
                        mstart(c + SD, b)

        plsc.subcore_barrier()
        pltpu.sync_copy(agg.at[pl.ds(sid * RPS, RPS)],
                        out_h.at[cid, pl.ds(sid * RPS, RPS)])

    return k(*msgs, dst2d, zeros)


def _final_body(p_ref, x_ref, bng_ref, bnb_ref, lng_ref, lnb_ref, out_ref):
    agg = p_ref[0, :N] + p_ref[1, :N]
    x = x_ref[...]
    out = agg * (1.0 / jnp.sqrt(1.0 + EPS)) * bng_ref[...] + bnb_ref[...]
    out = out + x
    mu = jnp.mean(out, axis=-1, keepdims=True)
    var = jnp.mean(jnp.square(out - mu), axis=-1, keepdims=True)
    out = (out - mu) * jax.lax.rsqrt(var + EPS) * lng_ref[...] + lnb_ref[...]
    out = jnp.maximum(out, 0.0)
    out_ref[...] = out + x


def _final(partial, x, bng, bnb, lng, lnb):
    return pl.pallas_call(
        _final_body,
        out_shape=jax.ShapeDtypeStruct((N, D), jnp.float32),
    )(partial, x, bng, bnb, lng, lnb)


def kernel(x, edge_index, edge_attr, W_f, b_f, W_s, b_s,
           bn_gamma, bn_beta, ln_gamma, ln_beta):
    src = edge_index[0]
    dst = edge_index[1]
    pad = E_PAD - E
    dst_p = jnp.pad(dst, (0, pad))
    src_p = jnp.pad(src, (0, pad))
    ea_p = jnp.pad(edge_attr, ((0, pad), (0, 0)))

    wd = jnp.concatenate([W_f[:D], W_s[:D]], axis=1)
    wsrc = jnp.concatenate([W_f[D:2 * D], W_s[D:2 * D]], axis=1)
    wfe = W_f[2 * D:]
    wse = W_s[2 * D:]

    tdst, tsrc = _project(x, wd, wsrc)
    bf2 = b_f.reshape(1, D)
    bs2 = b_s.reshape(1, D)
    abs_ = []
    for s in range(NSLICE):
        sl = slice(s * E_SL, (s + 1) * E_SL)
        abs_.append(_gather(tdst, tsrc, dst_p[sl], src_p[sl]))
    abs_ = jax.lax.optimization_barrier(abs_)
    msgs = []
    for s in range(NSLICE):
        sl = slice(s * E_SL, (s + 1) * E_SL)
        a, b = abs_[s]
        msgs.append(_mlp(s * E_SL, a, b, ea_p[sl], wfe, wse, bf2, bs2))
    dst2d = jnp.pad(dst_p, (0, 8 * GS)).reshape(E_PAD // GS + 8, GS)
    partial = _scatter(msgs, dst2d, jnp.zeros((N_PAD, D), jnp.float32))
    return _final(partial, x, bn_gamma.reshape(1, D), bn_beta.reshape(1, D),
                  ln_gamma.reshape(1, D), ln_beta.reshape(1, D))


# speedup vs baseline: 1.0392x; 1.0392x over previous
import functools

import jax
import jax.numpy as jnp
from jax import lax
from jax.experimental import pallas as pl
from jax.experimental.pallas import tpu as pltpu
from jax.experimental.pallas import tpu_sc as plsc

N = 10000
E = 320000
D = 128
D_EDGE = 16
EPS = 1e-5

NC = 2
NS = 16
NW = NC * NS
E_PAD = 327680
NSLICE = 2
E_SL = E_PAD // NSLICE
EPW = E_SL // NW
G = 64
NCH_G = EPW // G
DEPTH = 4
GS = 128
NCH_S = EPW // GS
N_PAD = 10240
RPS = N_PAD // NS

_MESH = plsc.VectorSubcoreMesh(core_axis_name="c", subcore_axis_name="s")


def _pack_bf16_pairs(t):
    bh = lax.bitcast_convert_type(t[:, :D], jnp.int32)
    bl = lax.bitcast_convert_type(t[:, D:], jnp.int32)
    rh = bh + jnp.int32(0x7FFF) + ((bh >> 16) & 1)
    rl = bl + jnp.int32(0x7FFF) + ((bl >> 16) & 1)
    return (rh & jnp.int32(-65536)) | lax.shift_right_logical(rl, 16)


def _unpack_hi(p):
    return lax.bitcast_convert_type(p & jnp.int32(-65536), jnp.float32)


def _unpack_lo(p):
    return lax.bitcast_convert_type(p << 16, jnp.float32)


def _proj_body(x_ref, wd_ref, ws_ref, tdst_ref, tsrc_ref):
    x = x_ref[...]
    tdst_ref[...] = _pack_bf16_pairs(
        jnp.dot(x, wd_ref[...], preferred_element_type=jnp.float32))
    tsrc_ref[...] = _pack_bf16_pairs(
        jnp.dot(x, ws_ref[...], preferred_element_type=jnp.float32))


def _project(x, wd, wsrc):
    return pl.pallas_call(
        _proj_body,
        out_shape=(
            jax.ShapeDtypeStruct((N, D), jnp.int32),
            jax.ShapeDtypeStruct((N, D), jnp.int32),
        ),
    )(x, wd, wsrc)


def _gather(tdst, tsrc, dst_idx, src_idx):
    out_ty = (
        jax.ShapeDtypeStruct((E_SL, D), jnp.int32),
        jax.ShapeDtypeStruct((E_SL, D), jnp.int32),
    )

    @functools.partial(
        pl.kernel,
        out_type=out_ty,
        mesh=_MESH,
        scratch_types=(
            [pltpu.VMEM((EPW,), jnp.int32)] * 2
            + [pltpu.VMEM((G, D), jnp.int32)] * (2 * DEPTH)
            + [pltpu.SemaphoreType.DMA] * (2 * DEPTH)
        ),
    )
    def k(tdst_h, tsrc_h, di_h, si_h, a_h, b_h, idxd, idxs, *rest):
        bd = list(rest[0:DEPTH])
        bs = list(rest[DEPTH:2 * DEPTH])
        gsem = list(rest[2 * DEPTH:3 * DEPTH])
        wsem = list(rest[3 * DEPTH:4 * DEPTH])
        cid = lax.axis_index("c")
        sid = lax.axis_index("s")
        wid = sid * NC + cid
        base = wid * EPW
        pltpu.sync_copy(di_h.at[pl.ds(base, EPW)], idxd)
        pltpu.sync_copy(si_h.at[pl.ds(base, EPW)], idxs)

        def gstart(c, b):
            pltpu.async_copy(tdst_h.at[idxd.at[pl.ds(c * G, G)]], bd[b], gsem[b])
            pltpu.async_copy(tsrc_h.at[idxs.at[pl.ds(c * G, G)]], bs[b], gsem[b])

        def gwait(c, b):
            pltpu.make_async_copy(
                tdst_h.at[idxd.at[pl.ds(c * G, G)]], bd[b], gsem[b]).wait()
            pltpu.make_async_copy(
                tsrc_h.at[idxs.at[pl.ds(c * G, G)]], bs[b], gsem[b]).wait()

        def wstart(c, b):
            pltpu.async_copy(bd[b], a_h.at[pl.ds(base + c * G, G)], wsem[b])
            pltpu.async_copy(bs[b], b_h.at[pl.ds(base + c * G, G)], wsem[b])

        def wwait(c, b):
            pltpu.make_async_copy(
                bd[b], a_h.at[pl.ds(base + c * G, G)], wsem[b]).wait()
            pltpu.make_async_copy(
                bs[b], b_h.at[pl.ds(base + c * G, G)], wsem[b]).wait()

        for b in range(DEPTH):
            gstart(b, b)

        @pl.loop(0, NCH_G, step=DEPTH)
        def _(cc):
            for b in range(DEPTH):
                c = cc + b
                gwait(c, b)
                wstart(c, b)

                @pl.when(c + DEPTH < NCH_G)
                def _():
                    wwait(c, b)
                    gstart(c + DEPTH, b)

        for b in range(DEPTH):
            wwait(NCH_G - DEPTH + b, b)

    return k(tdst, tsrc, dst_idx, src_idx)


BE = 2048


def _mlp_body(base, a_ref, b_ref, ea_ref, wfe_ref, wse_ref, bf_ref, bs_ref,
              msg_ref):
    a = a_ref[...]
    b = b_ref[...]
    ea = ea_ref[...]
    ef = jnp.dot(ea, wfe_ref[...], preferred_element_type=jnp.float32)
    es = jnp.dot(ea, wse_ref[...], preferred_element_type=jnp.float32)
    zf = _unpack_hi(a) + _unpack_hi(b) + ef + bf_ref[...]
    zs = _unpack_lo(a) + _unpack_lo(b) + es + bs_ref[...]
    gate = jax.nn.sigmoid(zf)
    val = jnp.maximum(zs, 0.0) + jnp.log1p(jnp.exp(-jnp.abs(zs)))
    msg = gate * val
    row = (base + pl.program_id(0) * BE
           + lax.broadcasted_iota(jnp.int32, (BE, D), 0))
    msg_ref[...] = jnp.where(row < E, msg, 0.0)


def _mlp(base, a, b, ea, wfe, wse, bf, bs):
    nblk = E_SL // BE
    return pl.pallas_call(
        functools.partial(_mlp_body, base),
        grid=(nblk,),
        in_specs=[
            pl.BlockSpec((BE, D), lambda i: (i, 0)),
            pl.BlockSpec((BE, D), lambda i: (i, 0)),
            pl.BlockSpec((BE, D_EDGE), lambda i: (i, 0)),
            pl.BlockSpec((D_EDGE, D), lambda i: (0, 0)),
            pl.BlockSpec((D_EDGE, D), lambda i: (0, 0)),
            pl.BlockSpec((1, D), lambda i: (0, 0)),
            pl.BlockSpec((1, D), lambda i: (0, 0)),
        ],
        out_specs=pl.BlockSpec((BE, D), lambda i: (i, 0)),
        out_shape=jax.ShapeDtypeStruct((E_SL, D), jnp.float32),
        compiler_params=pltpu.CompilerParams(
            dimension_semantics=("parallel",)),
    )(a, b, ea, wfe, wse, bf, bs)


def _scatter(msgs, dst2d, zeros):
    out_ty = jax.ShapeDtypeStruct((NC, N_PAD, D), jnp.float32)

    IW = ((NCH_S + 7) // 8) * 8 + 8

    @functools.partial(
        pl.kernel,
        out_type=out_ty,
        mesh=_MESH,
        scratch_types=[
            pltpu.VMEM((NSLICE * IW, GS), jnp.int32),
            pltpu.VMEM((GS, D), jnp.float32),
            pltpu.VMEM((GS, D), jnp.float32),
            pltpu.VMEM_SHARED((N_PAD, D), jnp.float32),
            pltpu.SemaphoreType.DMA,
            pltpu.SemaphoreType.DMA,
        ],
    )
    def k(*refs):
        msg_hs = list(refs[:NSLICE])
        (dst_h, z_h, out_h, idx2, m0, m1, agg, s0, s1) = refs[NSLICE:]
        cid = lax.axis_index("c")
        sid = lax.axis_index("s")
        wid = sid * NC + cid
        offs = []
        for s in range(NSLICE):
            gstart = s * (E_SL // GS) + wid * NCH_S
            astart = (gstart // 8) * 8
            offs.append(gstart - astart)
            pltpu.sync_copy(dst_h.at[pl.ds(astart, IW)],
                            idx2.at[pl.ds(s * IW, IW)])
        pltpu.sync_copy(z_h.at[pl.ds(sid * RPS, RPS)],
                        agg.at[pl.ds(sid * RPS, RPS)])
        plsc.subcore_barrier()

        mbuf = [m0, m1]
        sem = [s0, s1]
        SD = 2

        for s in range(NSLICE):
            msg_h = msg_hs[s]
            ebase = wid * EPW

            def mstart(c, b):
                pltpu.async_copy(
                    msg_h.at[pl.ds(ebase + c * GS, GS)], mbuf[b], sem[b])

            def mwait(c, b):
                pltpu.make_async_copy(
                    msg_h.at[pl.ds(ebase + c * GS, GS)], mbuf[b], sem[b]).wait()

            for b in range(SD):
                mstart(b, b)

            @pl.loop(0, NCH_S, step=SD)
            def _(cc):
                for b in range(SD):
                    c = cc + b
                    mwait(c, b)
                    pltpu.sync_copy(mbuf[b],
                                    agg.at[idx2.at[s * IW + offs[s] + c]],
                                    add=True)

                    @pl.when(c + SD < NCH_S)
                    def _():
                        mstart(c + SD, b)

        plsc.subcore_barrier()
        pltpu.sync_copy(agg.at[pl.ds(sid * RPS, RPS)],
                        out_h.at[cid, pl.ds(sid * RPS, RPS)])

    return k(*msgs, dst2d, zeros)


def _final_body(p_ref, x_ref, bng_ref, bnb_ref, lng_ref, lnb_ref, out_ref):
    agg = p_ref[0, :N] + p_ref[1, :N]
    x = x_ref[...]
    out = agg * (1.0 / jnp.sqrt(1.0 + EPS)) * bng_ref[...] + bnb_ref[...]
    out = out + x
    mu = jnp.mean(out, axis=-1, keepdims=True)
    var = jnp.mean(jnp.square(out - mu), axis=-1, keepdims=True)
    out = (out - mu) * jax.lax.rsqrt(var + EPS) * lng_ref[...] + lnb_ref[...]
    out = jnp.maximum(out, 0.0)
    out_ref[...] = out + x


def _final(partial, x, bng, bnb, lng, lnb):
    return pl.pallas_call(
        _final_body,
        out_shape=jax.ShapeDtypeStruct((N, D), jnp.float32),
    )(partial, x, bng, bnb, lng, lnb)


def kernel(x, edge_index, edge_attr, W_f, b_f, W_s, b_s,
           bn_gamma, bn_beta, ln_gamma, ln_beta):
    src = edge_index[0]
    dst = edge_index[1]
    pad = E_PAD - E
    dst_p = jnp.pad(dst, (0, pad))
    src_p = jnp.pad(src, (0, pad))
    ea_p = jnp.pad(edge_attr, ((0, pad), (0, 0)))

    wd = jnp.concatenate([W_f[:D], W_s[:D]], axis=1)
    wsrc = jnp.concatenate([W_f[D:2 * D], W_s[D:2 * D]], axis=1)
    wfe = W_f[2 * D:]
    wse = W_s[2 * D:]

    tdst, tsrc = _project(x, wd, wsrc)
    bf2 = b_f.reshape(1, D)
    bs2 = b_s.reshape(1, D)
    msgs = []
    for s in range(NSLICE):
        sl = slice(s * E_SL, (s + 1) * E_SL)
        a, b = _gather(tdst, tsrc, dst_p[sl], src_p[sl])
        msgs.append(_mlp(s * E_SL, a, b, ea_p[sl], wfe, wse, bf2, bs2))
    dst2d = jnp.pad(dst_p, (0, 8 * GS)).reshape(E_PAD // GS + 8, GS)
    partial = _scatter(msgs, dst2d, jnp.zeros((N_PAD, D), jnp.float32))
    return _final(partial, x, bn_gamma.reshape(1, D), bn_beta.reshape(1, D),
                  ln_gamma.reshape(1, D), ln_beta.reshape(1, D))
